# baseline (speedup 1.0000x reference)
import functools

import numpy as np
import jax
import jax.numpy as jnp
from jax import lax
from jax.experimental import pallas as pl
from jax.experimental.pallas import tpu as pltpu
from jax.experimental.pallas import tpu_sc as plsc

D_MODEL = 768
POS_LENGTH = 2048
BATCH = 4
SCALE = float(np.sqrt(float(D_MODEL)))

NUM_CORES = 2
NUM_SUBCORES = 16
NUM_WORKERS = NUM_CORES * NUM_SUBCORES
LANES = 16

ROWS_TOTAL = BATCH * POS_LENGTH
L_PER_W = POS_LENGTH // NUM_WORKERS
CHUNK_L = 8
N_CHUNKS = L_PER_W // CHUNK_L
VECS_PER_ROW = D_MODEL // LANES
N_RBUF = 3
N_PBUF = 2


def _positional_encoding(length: int, depth: int) -> np.ndarray:
    depth_half = depth / 2
    positions = np.arange(length)[:, np.newaxis].astype(np.float32)
    depths = (np.arange(depth_half)[np.newaxis, :] / depth_half).astype(np.float32)
    angle_rates = 1.0 / (10000.0 ** depths)
    angle_rads = positions * angle_rates
    return np.concatenate([np.sin(angle_rads), np.cos(angle_rads)], axis=-1)


_POS_NP = _positional_encoding(POS_LENGTH, D_MODEL).reshape(-1)


@functools.partial(
    pl.kernel,
    out_type=jax.ShapeDtypeStruct((ROWS_TOTAL, D_MODEL), jnp.float32),
    mesh=plsc.VectorSubcoreMesh(core_axis_name="c", subcore_axis_name="s"),
    scratch_types=(
        [pltpu.VMEM((BATCH, L_PER_W), jnp.int32)]
        + [pltpu.VMEM((BATCH * CHUNK_L, D_MODEL), jnp.float32)] * N_RBUF
        + [pltpu.VMEM((CHUNK_L * D_MODEL,), jnp.float32)] * N_PBUF
        + [pltpu.SemaphoreType.DMA] * (2 * N_RBUF + N_PBUF)
    ),
)
def _emb_kernel(table_hbm, idx_hbm, pos_hbm, out_hbm, idx_v,
                r0, r1, r2, p0, p1,
                gs0, gs1, gs2, os0, os1, os2, ps0, ps1):
    rows = [r0, r1, r2]
    posb = [p0, p1]
    gsem = [gs0, gs1, gs2]
    osem = [os0, os1, os2]
    psem = [ps0, ps1]

    wid = lax.axis_index("s") * NUM_CORES + lax.axis_index("c")
    l_base = wid * L_PER_W

    for b in range(BATCH):
        pltpu.sync_copy(
            idx_hbm.at[pl.ds(b * POS_LENGTH + l_base, L_PER_W)], idx_v.at[b]
        )

    def start_in(c):
        rb, pb = c % N_RBUF, c % N_PBUF
        gs = []
        for b in range(BATCH):
            gs.append(
                pltpu.async_copy(
                    table_hbm.at[idx_v.at[b, pl.ds(c * CHUNK_L, CHUNK_L)]],
                    rows[rb].at[pl.ds(b * CHUNK_L, CHUNK_L)],
                    gsem[rb],
                )
            )
        p = pltpu.async_copy(
            pos_hbm.at[pl.ds((l_base + c * CHUNK_L) * D_MODEL, CHUNK_L * D_MODEL)],
            posb[pb],
            psem[pb],
        )
        return gs, p

    def start_out(c):
        rb = c % N_RBUF
        hs = []
        for b in range(BATCH):
            hs.append(
                pltpu.async_copy(
                    rows[rb].at[pl.ds(b * CHUNK_L, CHUNK_L)],
                    out_hbm.at[pl.ds(b * POS_LENGTH + l_base + c * CHUNK_L, CHUNK_L)],
                    osem[rb],
                )
            )
        return hs

    in_fl = {0: start_in(0), 1: start_in(1)}
    out_fl = {}
    for c in range(N_CHUNKS):
        gs, p = in_fl.pop(c)
        for g in gs:
            g.wait()
        p.wait()
        if c - 1 in out_fl:
            for h in out_fl.pop(c - 1):
                h.wait()
        if c + 2 < N_CHUNKS:
            in_fl[c + 2] = start_in(c + 2)

        r_ref, p_ref = rows[c % N_RBUF], posb[c % N_PBUF]

        @plsc.parallel_loop(0, VECS_PER_ROW, 1, unroll=2)
        def vec_body(j):
            sl = pl.ds(j * LANES, LANES)
            for r in range(CHUNK_L):
                pv = p_ref[pl.ds(r * D_MODEL + j * LANES, LANES)]
                vals = [r_ref[b * CHUNK_L + r, sl] for b in range(BATCH)]
                res = [v * SCALE + pv for v in vals]
                for b in range(BATCH):
                    r_ref[b * CHUNK_L + r, sl] = res[b]

        out_fl[c] = start_out(c)

    for c in sorted(out_fl):
        for h in out_fl.pop(c):
            h.wait()


def kernel(x, table):
    b, l = x.shape
    idx = x.reshape(b * l).astype(jnp.int32)
    pos = jnp.asarray(_POS_NP, dtype=jnp.float32)
    out = _emb_kernel(table, idx, pos)
    return out.reshape(b, l, D_MODEL)


# speedup vs baseline: 1.0021x; 1.0021x over previous
import functools

import numpy as np
import jax
import jax.numpy as jnp
from jax import lax
from jax.experimental import pallas as pl
from jax.experimental.pallas import tpu as pltpu
from jax.experimental.pallas import tpu_sc as plsc

D_MODEL = 768
POS_LENGTH = 2048
BATCH = 4
SCALE = float(np.sqrt(float(D_MODEL)))

NUM_CORES = 2
NUM_SUBCORES = 16
NUM_WORKERS = NUM_CORES * NUM_SUBCORES
LANES = 16

ROWS_TOTAL = BATCH * POS_LENGTH
L_PER_W = POS_LENGTH // NUM_WORKERS
CHUNK_L = 8
N_CHUNKS = L_PER_W // CHUNK_L
VECS_PER_ROW = D_MODEL // LANES
N_RBUF = 3
N_PBUF = 3


def _positional_encoding(length: int, depth: int) -> np.ndarray:
    depth_half = depth / 2
    positions = np.arange(length)[:, np.newaxis].astype(np.float32)
    depths = (np.arange(depth_half)[np.newaxis, :] / depth_half).astype(np.float32)
    angle_rates = 1.0 / (10000.0 ** depths)
    angle_rads = positions * angle_rates
    return np.concatenate([np.sin(angle_rads), np.cos(angle_rads)], axis=-1)


_POS_NP = _positional_encoding(POS_LENGTH, D_MODEL).reshape(-1)


@functools.partial(
    pl.kernel,
    out_type=jax.ShapeDtypeStruct((ROWS_TOTAL, D_MODEL), jnp.float32),
    mesh=plsc.VectorSubcoreMesh(core_axis_name="c", subcore_axis_name="s"),
    scratch_types=(
        [pltpu.VMEM((BATCH, L_PER_W), jnp.int32)]
        + [pltpu.VMEM((BATCH * CHUNK_L, D_MODEL), jnp.float32)] * N_RBUF
        + [pltpu.VMEM((CHUNK_L * D_MODEL,), jnp.float32)] * N_PBUF
        + [pltpu.SemaphoreType.DMA] * (2 * N_RBUF + N_PBUF)
    ),
)
def _emb_kernel(table_hbm, idx_hbm, pos_hbm, out_hbm, idx_v,
                r0, r1, r2, p0, p1, p2,
                gs0, gs1, gs2, os0, os1, os2, ps0, ps1, ps2):
    rows = [r0, r1, r2]
    posb = [p0, p1, p2]
    gsem = [gs0, gs1, gs2]
    osem = [os0, os1, os2]
    psem = [ps0, ps1, ps2]

    wid = lax.axis_index("s") * NUM_CORES + lax.axis_index("c")
    l_base = wid * L_PER_W

    for b in range(BATCH):
        pltpu.sync_copy(
            idx_hbm.at[pl.ds(b * POS_LENGTH + l_base, L_PER_W)], idx_v.at[b]
        )

    def start_in(c):
        rb, pb = c % N_RBUF, c % N_PBUF
        gs = []
        for b in range(BATCH):
            gs.append(
                pltpu.async_copy(
                    table_hbm.at[idx_v.at[b, pl.ds(c * CHUNK_L, CHUNK_L)]],
                    rows[rb].at[pl.ds(b * CHUNK_L, CHUNK_L)],
                    gsem[rb],
                )
            )
        p = pltpu.async_copy(
            pos_hbm.at[pl.ds((l_base + c * CHUNK_L) * D_MODEL, CHUNK_L * D_MODEL)],
            posb[pb],
            psem[pb],
        )
        return gs, p

    def start_out(c):
        rb = c % N_RBUF
        hs = []
        for b in range(BATCH):
            hs.append(
                pltpu.async_copy(
                    rows[rb].at[pl.ds(b * CHUNK_L, CHUNK_L)],
                    out_hbm.at[pl.ds(b * POS_LENGTH + l_base + c * CHUNK_L, CHUNK_L)],
                    osem[rb],
                )
            )
        return hs

    in_fl = {0: start_in(0), 1: start_in(1)}
    out_fl = {}
    for c in range(N_CHUNKS):
        gs, p = in_fl.pop(c)
        for g in gs:
            g.wait()
        p.wait()
        if c - 1 in out_fl:
            for h in out_fl.pop(c - 1):
                h.wait()
        if c + 2 < N_CHUNKS:
            in_fl[c + 2] = start_in(c + 2)

        r_ref, p_ref = rows[c % N_RBUF], posb[c % N_PBUF]

        @plsc.parallel_loop(0, VECS_PER_ROW, 1, unroll=2)
        def vec_body(j):
            sl = pl.ds(j * LANES, LANES)
            for r in range(CHUNK_L):
                pv = p_ref[pl.ds(r * D_MODEL + j * LANES, LANES)]
                vals = [r_ref[b * CHUNK_L + r, sl] for b in range(BATCH)]
                res = [v * SCALE + pv for v in vals]
                for b in range(BATCH):
                    r_ref[b * CHUNK_L + r, sl] = res[b]

        out_fl[c] = start_out(c)

    for c in sorted(out_fl):
        for h in out_fl.pop(c):
            h.wait()


def kernel(x, table):
    b, l = x.shape
    idx = x.reshape(b * l).astype(jnp.int32)
    pos = jnp.asarray(_POS_NP, dtype=jnp.float32)
    out = _emb_kernel(table, idx, pos)
    return out.reshape(b, l, D_MODEL)


# speedup vs baseline: 1.0694x; 1.0671x over previous
import functools

import numpy as np
import jax
import jax.numpy as jnp
from jax import lax
from jax.experimental import pallas as pl
from jax.experimental.pallas import tpu as pltpu
from jax.experimental.pallas import tpu_sc as plsc

D_MODEL = 768
POS_LENGTH = 2048
BATCH = 4
SCALE = float(np.sqrt(float(D_MODEL)))

NUM_CORES = 2
NUM_SUBCORES = 16
NUM_WORKERS = NUM_CORES * NUM_SUBCORES
LANES = 16

ROWS_TOTAL = BATCH * POS_LENGTH
L_PER_W = POS_LENGTH // NUM_WORKERS
CHUNK_L = 16
N_CHUNKS = L_PER_W // CHUNK_L
VECS_PER_ROW = D_MODEL // LANES
N_RBUF = 2
N_PBUF = 2


def _positional_encoding(length: int, depth: int) -> np.ndarray:
    depth_half = depth / 2
    positions = np.arange(length)[:, np.newaxis].astype(np.float32)
    depths = (np.arange(depth_half)[np.newaxis, :] / depth_half).astype(np.float32)
    angle_rates = 1.0 / (10000.0 ** depths)
    angle_rads = positions * angle_rates
    return np.concatenate([np.sin(angle_rads), np.cos(angle_rads)], axis=-1)


_POS_NP = _positional_encoding(POS_LENGTH, D_MODEL)


@functools.partial(
    pl.kernel,
    out_type=jax.ShapeDtypeStruct((ROWS_TOTAL, D_MODEL), jnp.float32),
    mesh=plsc.VectorSubcoreMesh(core_axis_name="c", subcore_axis_name="s"),
    scratch_types=(
        [pltpu.VMEM((BATCH, L_PER_W), jnp.int32)]
        + [pltpu.VMEM((BATCH * CHUNK_L, D_MODEL), jnp.float32)] * N_RBUF
        + [pltpu.VMEM((CHUNK_L, D_MODEL), jnp.float32)] * N_PBUF
        + [pltpu.SemaphoreType.DMA] * (2 * N_RBUF + N_PBUF)
    ),
)
def _emb_kernel(table_hbm, idx_hbm, pos_hbm, out_hbm, idx_v,
                r0, r1, p0, p1,
                gs0, gs1, os0, os1, ps0, ps1):
    rows = [r0, r1]
    posb = [p0, p1]
    gsem = [gs0, gs1]
    osem = [os0, os1]
    psem = [ps0, ps1]

    wid = lax.axis_index("s") * NUM_CORES + lax.axis_index("c")
    l_base = wid * L_PER_W

    idx_handles = [
        pltpu.async_copy(
            idx_hbm.at[pl.ds(b * POS_LENGTH + l_base, L_PER_W)], idx_v.at[b], ps0
        )
        for b in range(BATCH)
    ]
    for h in idx_handles:
        h.wait()

    def start_in(c):
        rb, pb = c % N_RBUF, c % N_PBUF
        gs = []
        for b in range(BATCH):
            gs.append(
                pltpu.async_copy(
                    table_hbm.at[idx_v.at[b, pl.ds(c * CHUNK_L, CHUNK_L)]],
                    rows[rb].at[pl.ds(b * CHUNK_L, CHUNK_L)],
                    gsem[rb],
                )
            )
        p = pltpu.async_copy(
            pos_hbm.at[pl.ds(l_base + c * CHUNK_L, CHUNK_L)],
            posb[pb],
            psem[pb],
        )
        return gs, p

    def start_out(c):
        rb = c % N_RBUF
        hs = []
        for b in range(BATCH):
            hs.append(
                pltpu.async_copy(
                    rows[rb].at[pl.ds(b * CHUNK_L, CHUNK_L)],
                    out_hbm.at[pl.ds(b * POS_LENGTH + l_base + c * CHUNK_L, CHUNK_L)],
                    osem[rb],
                )
            )
        return hs

    in_fl = {0: start_in(0)}
    out_fl = {}
    for c in range(N_CHUNKS):
        gs, p = in_fl.pop(c)
        for g in gs:
            g.wait()
        p.wait()
        if c + 1 - N_RBUF in out_fl:
            for h in out_fl.pop(c + 1 - N_RBUF):
                h.wait()
        if c + 1 < N_CHUNKS:
            in_fl[c + 1] = start_in(c + 1)

        r_ref, p_ref = rows[c % N_RBUF], posb[c % N_PBUF]

        @plsc.parallel_loop(0, CHUNK_L, 1, unroll=1)
        def row_body(r):
            for j in range(VECS_PER_ROW):
                sl = pl.ds(j * LANES, LANES)
                pv = p_ref[r, sl]
                vals = [r_ref[b * CHUNK_L + r, sl] for b in range(BATCH)]
                res = [v * SCALE + pv for v in vals]
                for b in range(BATCH):
                    r_ref[b * CHUNK_L + r, sl] = res[b]

        out_fl[c] = start_out(c)

    for c in sorted(out_fl):
        for h in out_fl.pop(c):
            h.wait()


def kernel(x, table):
    b, l = x.shape
    idx = x.reshape(b * l).astype(jnp.int32)
    pos = jnp.asarray(_POS_NP, dtype=jnp.float32)
    out = _emb_kernel(table, idx, pos)
    return out.reshape(b, l, D_MODEL)


# speedup vs baseline: 1.0760x; 1.0061x over previous
import functools

import numpy as np
import jax
import jax.numpy as jnp
from jax import lax
from jax.experimental import pallas as pl
from jax.experimental.pallas import tpu as pltpu
from jax.experimental.pallas import tpu_sc as plsc

D_MODEL = 768
POS_LENGTH = 2048
BATCH = 4
SCALE = float(np.sqrt(float(D_MODEL)))

NUM_CORES = 2
NUM_SUBCORES = 16
NUM_WORKERS = NUM_CORES * NUM_SUBCORES
LANES = 16

ROWS_TOTAL = BATCH * POS_LENGTH
L_PER_W = POS_LENGTH // NUM_WORKERS
CHUNK_L = 16
N_CHUNKS = L_PER_W // CHUNK_L
VECS_PER_ROW = D_MODEL // LANES
N_RBUF = 2
N_PBUF = 2


def _positional_encoding(length: int, depth: int) -> np.ndarray:
    depth_half = depth / 2
    positions = np.arange(length)[:, np.newaxis].astype(np.float32)
    depths = (np.arange(depth_half)[np.newaxis, :] / depth_half).astype(np.float32)
    angle_rates = 1.0 / (10000.0 ** depths)
    angle_rads = positions * angle_rates
    return np.concatenate([np.sin(angle_rads), np.cos(angle_rads)], axis=-1)


_POS_NP = _positional_encoding(POS_LENGTH, D_MODEL)


@functools.partial(
    pl.kernel,
    out_type=jax.ShapeDtypeStruct((ROWS_TOTAL, D_MODEL), jnp.float32),
    mesh=plsc.VectorSubcoreMesh(core_axis_name="c", subcore_axis_name="s"),
    scratch_types=(
        [pltpu.VMEM((N_CHUNKS * BATCH * CHUNK_L,), jnp.int32)]
        + [pltpu.VMEM((BATCH * CHUNK_L, D_MODEL), jnp.float32)] * N_RBUF
        + [pltpu.VMEM((CHUNK_L, D_MODEL), jnp.float32)] * N_PBUF
        + [pltpu.SemaphoreType.DMA] * (2 * N_RBUF + N_PBUF)
    ),
)
def _emb_kernel(table_hbm, idx_hbm, pos_hbm, out_hbm, idx_v,
                r0, r1, p0, p1,
                gs0, gs1, os0, os1, ps0, ps1):
    rows = [r0, r1]
    posb = [p0, p1]
    gsem = [gs0, gs1]
    osem = [os0, os1]
    psem = [ps0, ps1]

    wid = lax.axis_index("s") * NUM_CORES + lax.axis_index("c")
    l_base = wid * L_PER_W

    idx_handles = [
        pltpu.async_copy(
            idx_hbm.at[pl.ds(b * POS_LENGTH + l_base + c * CHUNK_L, CHUNK_L)],
            idx_v.at[pl.ds((c * BATCH + b) * CHUNK_L, CHUNK_L)],
            ps0,
        )
        for c in range(N_CHUNKS)
        for b in range(BATCH)
    ]
    for h in idx_handles:
        h.wait()

    def start_in(c):
        rb, pb = c % N_RBUF, c % N_PBUF
        gs = [
            pltpu.async_copy(
                table_hbm.at[idx_v.at[pl.ds(c * BATCH * CHUNK_L, BATCH * CHUNK_L)]],
                rows[rb],
                gsem[rb],
            )
        ]
        p = pltpu.async_copy(
            pos_hbm.at[pl.ds(l_base + c * CHUNK_L, CHUNK_L)],
            posb[pb],
            psem[pb],
        )
        return gs, p

    def start_out(c):
        rb = c % N_RBUF
        hs = []
        for b in range(BATCH):
            hs.append(
                pltpu.async_copy(
                    rows[rb].at[pl.ds(b * CHUNK_L, CHUNK_L)],
                    out_hbm.at[pl.ds(b * POS_LENGTH + l_base + c * CHUNK_L, CHUNK_L)],
                    osem[rb],
                )
            )
        return hs

    in_fl = {0: start_in(0)}
    out_fl = {}
    for c in range(N_CHUNKS):
        gs, p = in_fl.pop(c)
        for g in gs:
            g.wait()
        p.wait()
        if c + 1 - N_RBUF in out_fl:
            for h in out_fl.pop(c + 1 - N_RBUF):
                h.wait()
        if c + 1 < N_CHUNKS:
            in_fl[c + 1] = start_in(c + 1)

        r_ref, p_ref = rows[c % N_RBUF], posb[c % N_PBUF]

        @plsc.parallel_loop(0, CHUNK_L, 1, unroll=1)
        def row_body(r):
            for j in range(VECS_PER_ROW):
                sl = pl.ds(j * LANES, LANES)
                pv = p_ref[r, sl]
                vals = [r_ref[b * CHUNK_L + r, sl] for b in range(BATCH)]
                res = [v * SCALE + pv for v in vals]
                for b in range(BATCH):
                    r_ref[b * CHUNK_L + r, sl] = res[b]

        out_fl[c] = start_out(c)

    for c in sorted(out_fl):
        for h in out_fl.pop(c):
            h.wait()


def kernel(x, table):
    b, l = x.shape
    idx = x.reshape(b * l).astype(jnp.int32)
    pos = jnp.asarray(_POS_NP, dtype=jnp.float32)
    out = _emb_kernel(table, idx, pos)
    return out.reshape(b, l, D_MODEL)
